# baseline (speedup 1.0000x reference)
import jax
import jax.numpy as jnp
from jax.experimental import pallas as pl
from jax.experimental.pallas import tpu as pltpu

TOPK = 8
NEXP = 64
HID = 4096
KSPLIT = 2
KCHUNK = HID // KSPLIT
BLOCK_T = 1024


def _gate_block(x_ref, w_ref, w_out_ref, i_out_ref, acc_ref):
    k = pl.program_id(1)
    x = x_ref[...]
    w = w_ref[:, pl.ds(k * KCHUNK, KCHUNK)]
    partial = jax.lax.dot_general(
        x, w, (((1,), (1,)), ((), ())),
        preferred_element_type=jnp.float32,
        precision=jax.lax.Precision.DEFAULT)

    @pl.when(k == 0)
    def _():
        acc_ref[...] = partial

    @pl.when(k == KSPLIT - 1)
    def _():
        logits = acc_ref[...] + partial
        lt = logits.T
        cmax = jnp.max(lt, axis=0, keepdims=True)
        denom = jnp.sum(jnp.exp(lt - cmax), axis=0, keepdims=True)
        iota = jax.lax.broadcasted_iota(jnp.int32, lt.shape, 0)
        work = lt
        vals, idxs = [], []
        for _ in range(TOPK):
            m = jnp.max(work, axis=0, keepdims=True)
            cand = jnp.where(work >= m, iota, NEXP)
            idx = jnp.min(cand, axis=0, keepdims=True)
            vals.append(m)
            idxs.append(idx)
            work = jnp.where(iota == idx, -jnp.inf, work)
        valcat = jnp.concatenate(vals, axis=0)
        idxcat = jnp.concatenate(idxs, axis=0)
        w_out_ref[...] = (jnp.exp(valcat - cmax) / denom).T
        i_out_ref[...] = idxcat.T


@jax.jit
def kernel(hidden_states, weight):
    h = hidden_states.shape[-1]
    x = hidden_states.reshape(-1, h).astype(jnp.float32)
    n_tok = x.shape[0]
    w = weight.astype(jnp.float32)
    grid = (n_tok // BLOCK_T, KSPLIT)
    w_out, i_out = pl.pallas_call(
        _gate_block,
        grid=grid,
        in_specs=[
            pl.BlockSpec((BLOCK_T, KCHUNK), lambda i, k: (i, k)),
            pl.BlockSpec((NEXP, HID), lambda i, k: (0, 0)),
        ],
        out_specs=[
            pl.BlockSpec((BLOCK_T, TOPK), lambda i, k: (i, 0)),
            pl.BlockSpec((BLOCK_T, TOPK), lambda i, k: (i, 0)),
        ],
        out_shape=[
            jax.ShapeDtypeStruct((n_tok, TOPK), jnp.float32),
            jax.ShapeDtypeStruct((n_tok, TOPK), jnp.int32),
        ],
        scratch_shapes=[pltpu.VMEM((BLOCK_T, NEXP), jnp.float32)],
        compiler_params=pltpu.CompilerParams(
            dimension_semantics=("arbitrary", "arbitrary"),
        ),
    )(x, w)
    return w_out, i_out


# speedup vs baseline: 1.1000x; 1.1000x over previous
import jax
import jax.numpy as jnp
from jax.experimental import pallas as pl
from jax.experimental.pallas import tpu as pltpu

TOPK = 8
NEXP = 64
HID = 4096
BLOCK_T = 1024


def _gate_block(x_ref, w_ref, w_out_ref, i_out_ref):
    x = x_ref[...]
    w = w_ref[...]
    logits = jax.lax.dot_general(
        x, w, (((1,), (1,)), ((), ())),
        preferred_element_type=jnp.float32,
        precision=jax.lax.Precision.DEFAULT)
    lt = logits.T
    cmax = jnp.max(lt, axis=0, keepdims=True)
    denom = jnp.sum(jnp.exp(lt - cmax), axis=0, keepdims=True)
    iota = jax.lax.broadcasted_iota(jnp.int32, lt.shape, 0)
    work = lt
    vals, idxs = [], []
    for _ in range(TOPK):
        m = jnp.max(work, axis=0, keepdims=True)
        cand = jnp.where(work >= m, iota, NEXP)
        idx = jnp.min(cand, axis=0, keepdims=True)
        vals.append(m)
        idxs.append(idx)
        work = jnp.where(iota == idx, -jnp.inf, work)
    valcat = jnp.concatenate(vals, axis=0)
    idxcat = jnp.concatenate(idxs, axis=0)
    w_out_ref[...] = (jnp.exp(valcat - cmax) / denom).T
    i_out_ref[...] = idxcat.T


@jax.jit
def kernel(hidden_states, weight):
    h = hidden_states.shape[-1]
    x = hidden_states.reshape(-1, h).astype(jnp.float32)
    n_tok = x.shape[0]
    w = weight.astype(jnp.float32)
    grid = (n_tok // BLOCK_T,)
    w_out, i_out = pl.pallas_call(
        _gate_block,
        grid=grid,
        in_specs=[
            pl.BlockSpec((BLOCK_T, HID), lambda i: (i, 0)),
            pl.BlockSpec((NEXP, HID), lambda i: (0, 0)),
        ],
        out_specs=[
            pl.BlockSpec((BLOCK_T, TOPK), lambda i: (i, 0)),
            pl.BlockSpec((BLOCK_T, TOPK), lambda i: (i, 0)),
        ],
        out_shape=[
            jax.ShapeDtypeStruct((n_tok, TOPK), jnp.float32),
            jax.ShapeDtypeStruct((n_tok, TOPK), jnp.int32),
        ],
        compiler_params=pltpu.CompilerParams(
            dimension_semantics=("arbitrary",),
        ),
    )(x, w)
    return w_out, i_out
